# baseline (speedup 1.0000x reference)
import functools

import jax
import jax.numpy as jnp
from jax.experimental import pallas as pl
from jax.experimental.pallas import tpu as pltpu

_BLOCK_N = 256
_LANES = 128
_N_CORES = 2


def _gather_ce_kernel(idx_sref, tgt_ref, table_ref, logits_ref, nll_ref,
                      tile_ref, *, block_n, n_pairs, stride, blocks_per_core):
    core = pl.program_id(0)
    j = pl.program_id(1)
    base = (core * blocks_per_core + j) * block_n

    for mi in range(block_n):
        s8 = pl.multiple_of(idx_sref[base + mi], n_pairs)
        tile_ref[mi:mi + n_pairs * stride:stride, :] = (
            table_ref[pl.ds(s8, n_pairs), :])

    for p in range(n_pairs):
        strip = tile_ref[pl.ds(p * stride, block_n), :]
        even = pltpu.bitcast(strip << 16, jnp.float32)
        odd = pltpu.bitcast(strip & jnp.int32(-65536), jnp.float32)
        logits_ref[:, (2 * p) * _LANES:(2 * p + 1) * _LANES] = even
        logits_ref[:, (2 * p + 1) * _LANES:(2 * p + 2) * _LANES] = odd

    logits = logits_ref[...]
    V = logits.shape[1]
    m = jnp.max(logits, axis=-1, keepdims=True)
    lse = m + jnp.log(jnp.sum(jnp.exp(logits - m), axis=-1, keepdims=True))
    col = jax.lax.broadcasted_iota(jnp.int32, (block_n, V), 1)
    tgt = tgt_ref[...]
    tgt_logit = jnp.sum(jnp.where(col == tgt, logits, 0.0),
                        axis=-1, keepdims=True)
    nll_ref[...] = lse - tgt_logit


def _bigram_forward(idx, table, targets, *, block_n=_BLOCK_N):
    B, T = idx.shape
    V = table.shape[0]
    N = B * T
    n_pairs = V // (2 * _LANES)
    num_blocks = N // block_n
    blocks_per_core = num_blocks // _N_CORES
    stride = block_n + 8
    tile_rows = (n_pairs - 1) * stride + block_n

    bits = jax.lax.bitcast_convert_type(
        table.astype(jnp.bfloat16), jnp.uint16).astype(jnp.uint32)
    bits = bits.reshape(V, n_pairs, 2, _LANES)
    words = (bits[:, :, 0, :] | (bits[:, :, 1, :] << 16)).astype(jnp.int32)
    table_packed = words.reshape(V * n_pairs, _LANES)

    idx_scaled = (idx.astype(jnp.int32) * n_pairs).reshape(N)
    tgt_col = targets.astype(jnp.int32).reshape(N, 1)

    kern = functools.partial(_gather_ce_kernel, block_n=block_n,
                             n_pairs=n_pairs, stride=stride,
                             blocks_per_core=blocks_per_core)

    def _blk(i, j, s):
        return (i * blocks_per_core + j, 0)

    logits_flat, nll = pl.pallas_call(
        kern,
        grid_spec=pltpu.PrefetchScalarGridSpec(
            num_scalar_prefetch=1,
            grid=(_N_CORES, blocks_per_core),
            in_specs=[
                pl.BlockSpec((block_n, 1), _blk),
                pl.BlockSpec((V * n_pairs, _LANES), lambda i, j, s: (0, 0)),
            ],
            out_specs=(
                pl.BlockSpec((block_n, V), _blk),
                pl.BlockSpec((block_n, 1), _blk),
            ),
            scratch_shapes=[pltpu.VMEM((tile_rows, _LANES), jnp.int32)],
        ),
        out_shape=(
            jax.ShapeDtypeStruct((N, V), jnp.float32),
            jax.ShapeDtypeStruct((N, 1), jnp.float32),
        ),
        compiler_params=pltpu.CompilerParams(
            dimension_semantics=("parallel", "arbitrary"),
            vmem_limit_bytes=32 * 1024 * 1024,
        ),
        cost_estimate=pl.CostEstimate(
            flops=4 * N * V,
            transcendentals=N * V,
            bytes_accessed=N * V * 4 + N * V * 2 + V * V * 2,
        ),
    )(idx_scaled, tgt_col, table_packed)

    logits = logits_flat.reshape(B, T, V)
    loss = jnp.sum(nll[:, 0]) / N
    return logits, loss


def kernel(idx, table, targets):
    return _bigram_forward(idx, table, targets)


# speedup vs baseline: 1.6340x; 1.6340x over previous
import functools

import jax
import jax.numpy as jnp
from jax.experimental import pallas as pl
from jax.experimental.pallas import tpu as pltpu

_BLOCK_N = 256
_LANES = 128
_N_CORES = 2


def _gather_ce_kernel(idx_sref, tgt_ref, tbl_ref, logits_ref, nll_ref,
                      rowmaj_ref, stage_ref, tile_ref, *, block_n, n_chunks,
                      stride, blocks_per_core, n_slices):
    core = pl.program_id(0)
    j = pl.program_id(1)
    slice_rows = block_n * n_chunks

    @pl.when(j < n_slices)
    def _relayout_slice():
        for c in range(n_chunks):
            stage_ref[c:c + n_chunks * block_n:n_chunks, :] = (
                tbl_ref[:, c * _LANES:(c + 1) * _LANES])
        rowmaj_ref[pl.ds(j * slice_rows, slice_rows), :] = stage_ref[...]

    @pl.when(j >= n_slices)
    def _gather_block():
        base = (core * blocks_per_core + (j - n_slices)) * block_n

        for mi in range(block_n):
            i16 = pl.multiple_of(idx_sref[base + mi], n_chunks)
            tile_ref[mi:mi + n_chunks * stride:stride, :] = (
                rowmaj_ref[pl.ds(i16, n_chunks), :])

        for c in range(n_chunks):
            logits_ref[:, c * _LANES:(c + 1) * _LANES] = (
                tile_ref[pl.ds(c * stride, block_n), :])

        logits = logits_ref[...]
        V = logits.shape[1]
        m = jnp.max(logits, axis=-1, keepdims=True)
        lse = m + jnp.log(jnp.sum(jnp.exp(logits - m), axis=-1,
                                  keepdims=True))
        col = jax.lax.broadcasted_iota(jnp.int32, (block_n, V), 1)
        tgt = tgt_ref[...]
        tgt_logit = jnp.sum(jnp.where(col == tgt, logits, 0.0),
                            axis=-1, keepdims=True)
        nll_ref[...] = lse - tgt_logit


def _bigram_forward(idx, table, targets, *, block_n=_BLOCK_N):
    B, T = idx.shape
    V = table.shape[0]
    N = B * T
    n_chunks = V // _LANES
    num_blocks = N // block_n
    blocks_per_core = num_blocks // _N_CORES
    n_slices = V // block_n
    n_steps = n_slices + blocks_per_core
    stride = block_n + 8
    tile_rows = (n_chunks - 1) * stride + block_n

    idx_scaled = (idx.astype(jnp.int32) * n_chunks).reshape(N)
    tgt_col = targets.astype(jnp.int32).reshape(N, 1)

    kern = functools.partial(_gather_ce_kernel, block_n=block_n,
                             n_chunks=n_chunks, stride=stride,
                             blocks_per_core=blocks_per_core,
                             n_slices=n_slices)

    def _out_blk(i, j, s):
        return (i * blocks_per_core + jnp.maximum(j - n_slices, 0), 0)

    logits_flat, nll = pl.pallas_call(
        kern,
        grid_spec=pltpu.PrefetchScalarGridSpec(
            num_scalar_prefetch=1,
            grid=(_N_CORES, n_steps),
            in_specs=[
                pl.BlockSpec((block_n, 1), _out_blk),
                pl.BlockSpec((block_n, V),
                             lambda i, j, s: (jnp.minimum(j, n_slices - 1), 0)),
            ],
            out_specs=(
                pl.BlockSpec((block_n, V), _out_blk),
                pl.BlockSpec((block_n, 1), _out_blk),
            ),
            scratch_shapes=[
                pltpu.VMEM((V * n_chunks, _LANES), jnp.float32),
                pltpu.VMEM((block_n * n_chunks, _LANES), jnp.float32),
                pltpu.VMEM((tile_rows, _LANES), jnp.float32),
            ],
        ),
        out_shape=(
            jax.ShapeDtypeStruct((N, V), jnp.float32),
            jax.ShapeDtypeStruct((N, 1), jnp.float32),
        ),
        compiler_params=pltpu.CompilerParams(
            dimension_semantics=("parallel", "arbitrary"),
            vmem_limit_bytes=40 * 1024 * 1024,
        ),
        cost_estimate=pl.CostEstimate(
            flops=4 * N * V,
            transcendentals=N * V,
            bytes_accessed=N * V * 4 * 2 + V * V * 4,
        ),
    )(idx_scaled, tgt_col, table)

    logits = logits_flat.reshape(B, T, V)
    loss = jnp.sum(nll[:, 0]) / N
    return logits, loss


def kernel(idx, table, targets):
    return _bigram_forward(idx, table, targets)


# speedup vs baseline: 2.1673x; 1.3263x over previous
import functools

import jax
import jax.numpy as jnp
from jax.experimental import pallas as pl
from jax.experimental.pallas import tpu as pltpu

_BLOCK_N = 256
_LANES = 128


def _gather_ce_kernel(idx_sref, tgt_ref, tbl_ref, logits_ref, nll_ref,
                      rowmaj_ref, stage_ref, tile_ref, *, block_n, n_chunks,
                      stride, n_slices):
    j = pl.program_id(0)
    slice_rows = block_n * n_chunks

    @pl.when(j < n_slices)
    def _relayout_slice():
        for c in range(n_chunks):
            stage_ref[c:c + n_chunks * block_n:n_chunks, :] = (
                tbl_ref[:, c * _LANES:(c + 1) * _LANES])
        rowmaj_ref[pl.ds(j * slice_rows, slice_rows), :] = stage_ref[...]

    @pl.when(j >= n_slices)
    def _gather_block():
        base = (j - n_slices) * block_n

        for mi in range(block_n):
            i16 = pl.multiple_of(idx_sref[base + mi], n_chunks)
            tile_ref[mi:mi + n_chunks * stride:stride, :] = (
                rowmaj_ref[pl.ds(i16, n_chunks), :])

        for c in range(n_chunks):
            logits_ref[:, c * _LANES:(c + 1) * _LANES] = (
                tile_ref[pl.ds(c * stride, block_n), :])

        logits = logits_ref[...]
        V = logits.shape[1]
        m = jnp.max(logits, axis=-1, keepdims=True)
        lse = m + jnp.log(jnp.sum(jnp.exp(logits - m), axis=-1,
                                  keepdims=True))
        col = jax.lax.broadcasted_iota(jnp.int32, (block_n, V), 1)
        tgt = tgt_ref[...]
        tgt_logit = jnp.sum(jnp.where(col == tgt, logits, 0.0),
                            axis=-1, keepdims=True)
        nll_ref[...] = lse - tgt_logit


def _bigram_forward(idx, table, targets, *, block_n=_BLOCK_N):
    B, T = idx.shape
    V = table.shape[0]
    N = B * T
    n_chunks = V // _LANES
    num_blocks = N // block_n
    n_slices = V // block_n
    n_steps = n_slices + num_blocks
    stride = block_n + 8
    tile_rows = (n_chunks - 1) * stride + block_n

    idx_scaled = (idx.astype(jnp.int32) * n_chunks).reshape(N)
    tgt_col = targets.astype(jnp.int32).reshape(N, 1)

    kern = functools.partial(_gather_ce_kernel, block_n=block_n,
                             n_chunks=n_chunks, stride=stride,
                             n_slices=n_slices)

    def _out_blk(j, s):
        return (jnp.maximum(j - n_slices, 0), 0)

    logits_flat, nll = pl.pallas_call(
        kern,
        grid_spec=pltpu.PrefetchScalarGridSpec(
            num_scalar_prefetch=1,
            grid=(n_steps,),
            in_specs=[
                pl.BlockSpec((block_n, 1), _out_blk),
                pl.BlockSpec((block_n, V),
                             lambda j, s: (jnp.minimum(j, n_slices - 1), 0)),
            ],
            out_specs=(
                pl.BlockSpec((block_n, V), _out_blk),
                pl.BlockSpec((block_n, 1), _out_blk),
            ),
            scratch_shapes=[
                pltpu.VMEM((V * n_chunks, _LANES), jnp.float32),
                pltpu.VMEM((block_n * n_chunks, _LANES), jnp.float32),
                pltpu.VMEM((tile_rows, _LANES), jnp.float32),
            ],
        ),
        out_shape=(
            jax.ShapeDtypeStruct((N, V), jnp.float32),
            jax.ShapeDtypeStruct((N, 1), jnp.float32),
        ),
        compiler_params=pltpu.CompilerParams(
            dimension_semantics=("arbitrary",),
            vmem_limit_bytes=40 * 1024 * 1024,
        ),
        cost_estimate=pl.CostEstimate(
            flops=4 * N * V,
            transcendentals=N * V,
            bytes_accessed=N * V * 4 * 2 + V * V * 4,
        ),
    )(idx_scaled, tgt_col, table)

    logits = logits_flat.reshape(B, T, V)
    loss = jnp.sum(nll[:, 0]) / N
    return logits, loss


def kernel(idx, table, targets):
    return _bigram_forward(idx, table, targets)


# speedup vs baseline: 2.1740x; 1.0031x over previous
import functools

import jax
import jax.numpy as jnp
from jax.experimental import pallas as pl
from jax.experimental.pallas import tpu as pltpu

_BLOCK_N = 256
_LANES = 128


def _gather_ce_kernel(idx_sref, tgt_ref, tbl_ref, logits_ref, loss_ref,
                      rowmaj_ref, stage_ref, tile_ref, *, block_n, n_chunks,
                      stride, n_slices, n_tokens):
    j = pl.program_id(0)
    slice_rows = block_n * n_chunks

    @pl.when(j < n_slices)
    def _relayout_slice():
        for c in range(n_chunks):
            stage_ref[c:c + n_chunks * block_n:n_chunks, :] = (
                tbl_ref[:, c * _LANES:(c + 1) * _LANES])
        rowmaj_ref[pl.ds(j * slice_rows, slice_rows), :] = stage_ref[...]

    @pl.when(j >= n_slices)
    def _gather_block():
        b = j - n_slices

        for mi in range(block_n):
            i16 = pl.multiple_of(idx_sref[b, mi] * n_chunks, n_chunks)
            tile_ref[mi:mi + n_chunks * stride:stride, :] = (
                rowmaj_ref[pl.ds(i16, n_chunks), :])

        for c in range(n_chunks):
            logits_ref[:, c * _LANES:(c + 1) * _LANES] = (
                tile_ref[pl.ds(c * stride, block_n), :])

        logits = logits_ref[...]
        V = logits.shape[1]
        m = jnp.max(logits, axis=-1, keepdims=True)
        lse = m + jnp.log(jnp.sum(jnp.exp(logits - m), axis=-1,
                                  keepdims=True))
        col = jax.lax.broadcasted_iota(jnp.int32, (block_n, V), 1)
        tgt = tgt_ref[...]
        tgt_logit = jnp.sum(jnp.where(col == tgt, logits, 0.0),
                            axis=-1, keepdims=True)
        block_loss = (jnp.sum(lse - tgt_logit) / n_tokens)[None, None]

        @pl.when(j == n_slices)
        def _init():
            loss_ref[...] = block_loss

        @pl.when(j > n_slices)
        def _accum():
            loss_ref[...] = loss_ref[...] + block_loss


def _bigram_forward(idx, table, targets, *, block_n=_BLOCK_N):
    B, T = idx.shape
    V = table.shape[0]
    N = B * T
    n_chunks = V // _LANES
    num_blocks = N // block_n
    n_slices = V // block_n
    n_steps = n_slices + num_blocks
    stride = block_n + 8
    tile_rows = (n_chunks - 1) * stride + block_n

    tgt_col = targets.astype(jnp.int32).reshape(N, 1)

    kern = functools.partial(_gather_ce_kernel, block_n=block_n,
                             n_chunks=n_chunks, stride=stride,
                             n_slices=n_slices, n_tokens=N)

    def _out_blk(j, s):
        return (jnp.maximum(j - n_slices, 0), 0)

    logits_flat, loss_arr = pl.pallas_call(
        kern,
        grid_spec=pltpu.PrefetchScalarGridSpec(
            num_scalar_prefetch=1,
            grid=(n_steps,),
            in_specs=[
                pl.BlockSpec((block_n, 1), _out_blk),
                pl.BlockSpec((block_n, V),
                             lambda j, s: (jnp.minimum(j, n_slices - 1), 0)),
            ],
            out_specs=(
                pl.BlockSpec((block_n, V), _out_blk),
                pl.BlockSpec((1, 1), lambda j, s: (0, 0)),
            ),
            scratch_shapes=[
                pltpu.VMEM((V * n_chunks, _LANES), jnp.float32),
                pltpu.VMEM((block_n * n_chunks, _LANES), jnp.float32),
                pltpu.VMEM((tile_rows, _LANES), jnp.float32),
            ],
        ),
        out_shape=(
            jax.ShapeDtypeStruct((N, V), jnp.float32),
            jax.ShapeDtypeStruct((1, 1), jnp.float32),
        ),
        compiler_params=pltpu.CompilerParams(
            dimension_semantics=("arbitrary",),
            vmem_limit_bytes=40 * 1024 * 1024,
        ),
        cost_estimate=pl.CostEstimate(
            flops=4 * N * V,
            transcendentals=N * V,
            bytes_accessed=N * V * 4 * 2 + V * V * 4,
        ),
    )(idx.astype(jnp.int32), tgt_col, table)

    logits = logits_flat.reshape(B, T, V)
    loss = loss_arr[0, 0]
    return logits, loss


def kernel(idx, table, targets):
    return _bigram_forward(idx, table, targets)


# speedup vs baseline: 2.4930x; 1.1468x over previous
import functools

import jax
import jax.numpy as jnp
from jax.experimental import pallas as pl
from jax.experimental.pallas import tpu as pltpu

_BLOCK_N = 256
_LANES = 128


def _gather_ce_kernel(idx_sref, tgt_ref, tbl_ref, logits_ref, loss_ref,
                      rowmaj_ref, stage_ref, tile_ref, *, block_n, n_pairs,
                      stride, n_slices, n_tokens):
    j = pl.program_id(0)
    slice_rows = block_n * n_pairs

    @pl.when(j < n_slices)
    def _relayout_slice():
        for p in range(n_pairs):
            e = pltpu.bitcast(tbl_ref[:, (2 * p) * _LANES:
                                      (2 * p + 1) * _LANES], jnp.uint32)
            o = pltpu.bitcast(tbl_ref[:, (2 * p + 1) * _LANES:
                                      (2 * p + 2) * _LANES], jnp.uint32)
            half = jnp.uint32(0x7FFF)
            one = jnp.uint32(1)
            lo = ((e + half + ((e >> 16) & one)) >> 16) & jnp.uint32(0xFFFF)
            hi = (o + half + ((o >> 16) & one)) & jnp.uint32(0xFFFF0000)
            stage_ref[p:p + n_pairs * block_n:n_pairs, :] = lo | hi
        rowmaj_ref[pl.ds(j * slice_rows, slice_rows), :] = stage_ref[...]

    @pl.when(j >= n_slices)
    def _gather_block():
        b = j - n_slices

        for mi in range(block_n):
            s8 = pl.multiple_of(idx_sref[b, mi] * n_pairs, n_pairs)
            tile_ref[mi:mi + n_pairs * stride:stride, :] = (
                rowmaj_ref[pl.ds(s8, n_pairs), :])

        for p in range(n_pairs):
            strip = tile_ref[pl.ds(p * stride, block_n), :]
            even = pltpu.bitcast(strip << 16, jnp.float32)
            odd = pltpu.bitcast(strip & jnp.uint32(0xFFFF0000), jnp.float32)
            logits_ref[:, (2 * p) * _LANES:(2 * p + 1) * _LANES] = even
            logits_ref[:, (2 * p + 1) * _LANES:(2 * p + 2) * _LANES] = odd

        logits = logits_ref[...]
        V = logits.shape[1]
        m = jnp.max(logits, axis=-1, keepdims=True)
        lse = m + jnp.log(jnp.sum(jnp.exp(logits - m), axis=-1,
                                  keepdims=True))
        col = jax.lax.broadcasted_iota(jnp.int32, (block_n, V), 1)
        tgt = tgt_ref[...]
        tgt_logit = jnp.sum(jnp.where(col == tgt, logits, 0.0),
                            axis=-1, keepdims=True)
        block_loss = (jnp.sum(lse - tgt_logit) / n_tokens)[None, None]

        @pl.when(j == n_slices)
        def _init():
            loss_ref[...] = block_loss

        @pl.when(j > n_slices)
        def _accum():
            loss_ref[...] = loss_ref[...] + block_loss


def _bigram_forward(idx, table, targets, *, block_n=_BLOCK_N):
    B, T = idx.shape
    V = table.shape[0]
    N = B * T
    n_pairs = V // (2 * _LANES)
    num_blocks = N // block_n
    n_slices = V // block_n
    n_steps = n_slices + num_blocks
    stride = block_n + 8
    tile_rows = (n_pairs - 1) * stride + block_n

    tgt_col = targets.astype(jnp.int32).reshape(N, 1)

    kern = functools.partial(_gather_ce_kernel, block_n=block_n,
                             n_pairs=n_pairs, stride=stride,
                             n_slices=n_slices, n_tokens=N)

    def _out_blk(j, s):
        return (jnp.maximum(j - n_slices, 0), 0)

    logits_flat, loss_arr = pl.pallas_call(
        kern,
        grid_spec=pltpu.PrefetchScalarGridSpec(
            num_scalar_prefetch=1,
            grid=(n_steps,),
            in_specs=[
                pl.BlockSpec((block_n, 1), _out_blk),
                pl.BlockSpec((block_n, V),
                             lambda j, s: (jnp.minimum(j, n_slices - 1), 0)),
            ],
            out_specs=(
                pl.BlockSpec((block_n, V), _out_blk),
                pl.BlockSpec((1, 1), lambda j, s: (0, 0)),
            ),
            scratch_shapes=[
                pltpu.VMEM((V * n_pairs, _LANES), jnp.uint32),
                pltpu.VMEM((block_n * n_pairs, _LANES), jnp.uint32),
                pltpu.VMEM((tile_rows, _LANES), jnp.uint32),
            ],
        ),
        out_shape=(
            jax.ShapeDtypeStruct((N, V), jnp.float32),
            jax.ShapeDtypeStruct((1, 1), jnp.float32),
        ),
        compiler_params=pltpu.CompilerParams(
            dimension_semantics=("arbitrary",),
            vmem_limit_bytes=40 * 1024 * 1024,
        ),
        cost_estimate=pl.CostEstimate(
            flops=4 * N * V,
            transcendentals=N * V,
            bytes_accessed=N * V * 4 * 2 + V * V * 4,
        ),
    )(idx.astype(jnp.int32), tgt_col, table)

    logits = logits_flat.reshape(B, T, V)
    loss = loss_arr[0, 0]
    return logits, loss


def kernel(idx, table, targets):
    return _bigram_forward(idx, table, targets)


# speedup vs baseline: 2.6871x; 1.0778x over previous
import functools

import jax
import jax.numpy as jnp
from jax.experimental import pallas as pl
from jax.experimental.pallas import tpu as pltpu

_BLOCK_N = 512
_LANES = 128


def _gather_ce_kernel(idx_sref, tgt_ref, tbl_ref, logits_ref, loss_ref,
                      rowmaj_ref, stage_ref, tile_ref, *, block_n, n_pairs,
                      stride, n_slices, n_tokens):
    j = pl.program_id(0)
    slice_rows = block_n * n_pairs

    @pl.when(j < n_slices)
    def _relayout_slice():
        for p in range(n_pairs):
            e = pltpu.bitcast(tbl_ref[:, (2 * p) * _LANES:
                                      (2 * p + 1) * _LANES], jnp.uint32)
            o = pltpu.bitcast(tbl_ref[:, (2 * p + 1) * _LANES:
                                      (2 * p + 2) * _LANES], jnp.uint32)
            half = jnp.uint32(0x7FFF)
            one = jnp.uint32(1)
            lo = ((e + half + ((e >> 16) & one)) >> 16) & jnp.uint32(0xFFFF)
            hi = (o + half + ((o >> 16) & one)) & jnp.uint32(0xFFFF0000)
            stage_ref[p:p + n_pairs * block_n:n_pairs, :] = lo | hi
        rowmaj_ref[pl.ds(j * slice_rows, slice_rows), :] = stage_ref[...]

    @pl.when(j >= n_slices)
    def _gather_block():
        b = j - n_slices

        for mi in range(block_n):
            s8 = pl.multiple_of(idx_sref[b, mi] * n_pairs, n_pairs)
            tile_ref[mi:mi + n_pairs * stride:stride, :] = (
                rowmaj_ref[pl.ds(s8, n_pairs), :])

        for p in range(n_pairs):
            strip = tile_ref[pl.ds(p * stride, block_n), :]
            even = pltpu.bitcast(strip << 16, jnp.float32)
            odd = pltpu.bitcast(strip & jnp.uint32(0xFFFF0000), jnp.float32)
            logits_ref[:, (2 * p) * _LANES:(2 * p + 1) * _LANES] = even
            logits_ref[:, (2 * p + 1) * _LANES:(2 * p + 2) * _LANES] = odd

        logits = logits_ref[...]
        V = logits.shape[1]
        m = jnp.max(logits, axis=-1, keepdims=True)
        lse = m + jnp.log(jnp.sum(jnp.exp(logits - m), axis=-1,
                                  keepdims=True))
        col = jax.lax.broadcasted_iota(jnp.int32, (block_n, V), 1)
        tgt = tgt_ref[...]
        tgt_logit = jnp.sum(jnp.where(col == tgt, logits, 0.0),
                            axis=-1, keepdims=True)
        block_loss = (jnp.sum(lse - tgt_logit) / n_tokens)[None, None]

        @pl.when(j == n_slices)
        def _init():
            loss_ref[...] = block_loss

        @pl.when(j > n_slices)
        def _accum():
            loss_ref[...] = loss_ref[...] + block_loss


def _bigram_forward(idx, table, targets, *, block_n=_BLOCK_N):
    B, T = idx.shape
    V = table.shape[0]
    N = B * T
    n_pairs = V // (2 * _LANES)
    num_blocks = N // block_n
    n_slices = V // block_n
    n_steps = n_slices + num_blocks
    stride = block_n + 8
    tile_rows = (n_pairs - 1) * stride + block_n

    tgt_col = targets.astype(jnp.int32).reshape(N, 1)

    kern = functools.partial(_gather_ce_kernel, block_n=block_n,
                             n_pairs=n_pairs, stride=stride,
                             n_slices=n_slices, n_tokens=N)

    def _out_blk(j, s):
        return (jnp.maximum(j - n_slices, 0), 0)

    logits_flat, loss_arr = pl.pallas_call(
        kern,
        grid_spec=pltpu.PrefetchScalarGridSpec(
            num_scalar_prefetch=1,
            grid=(n_steps,),
            in_specs=[
                pl.BlockSpec((block_n, 1), _out_blk),
                pl.BlockSpec((block_n, V),
                             lambda j, s: (jnp.minimum(j, n_slices - 1), 0)),
            ],
            out_specs=(
                pl.BlockSpec((block_n, V), _out_blk),
                pl.BlockSpec((1, 1), lambda j, s: (0, 0)),
            ),
            scratch_shapes=[
                pltpu.VMEM((V * n_pairs, _LANES), jnp.uint32),
                pltpu.VMEM((block_n * n_pairs, _LANES), jnp.uint32),
                pltpu.VMEM((tile_rows, _LANES), jnp.uint32),
            ],
        ),
        out_shape=(
            jax.ShapeDtypeStruct((N, V), jnp.float32),
            jax.ShapeDtypeStruct((1, 1), jnp.float32),
        ),
        compiler_params=pltpu.CompilerParams(
            dimension_semantics=("arbitrary",),
            vmem_limit_bytes=40 * 1024 * 1024,
        ),
        cost_estimate=pl.CostEstimate(
            flops=4 * N * V,
            transcendentals=N * V,
            bytes_accessed=N * V * 4 * 2 + V * V * 4,
        ),
    )(idx.astype(jnp.int32), tgt_col, table)

    logits = logits_flat.reshape(B, T, V)
    loss = loss_arr[0, 0]
    return logits, loss


def kernel(idx, table, targets):
    return _bigram_forward(idx, table, targets)
